# Initial kernel scaffold; baseline (speedup 1.0000x reference)
import jax
import jax.numpy as jnp
from jax.experimental import pallas as pl


def kernel(pixel_values, patch_coords, cls_token, W_in, W_coord, W_q, W_k):
    raise NotImplementedError("write your pallas kernel here")


# speedup vs baseline: 2424.7360x; 2424.7360x over previous
import math

import jax
import jax.numpy as jnp
from jax.experimental import pallas as pl
from jax.experimental.pallas import tpu as pltpu

_B, _P, _D = 4, 4096, 768
_T = 256
_KPT = 32
_NEG = -1e30
_CH = 128
_MT = 1024
_TCH = 128
_CK = 1024


def _dot_t(a, b):
    return jax.lax.dot_general(a, b, (((1,), (1,)), ((), ())),
                               preferred_element_type=jnp.float32)


def _dot_x(a, b):
    return jax.lax.dot_general(a, b, (((1,), (0,)), ((), ())),
                               preferred_element_type=jnp.float32,
                               precision=jax.lax.Precision.HIGHEST)


def _to_col(v_row):
    ones = jnp.ones((1, 1), jnp.float32)
    return jax.lax.dot_general(v_row, ones, (((0,), (0,)), ((), ())),
                               preferred_element_type=jnp.float32,
                               precision=jax.lax.Precision.HIGHEST)


def _sort_key(x):
    u = jax.lax.bitcast_convert_type(x, jnp.uint32)
    neg = (u >> jnp.uint32(31)) == jnp.uint32(1)
    return jnp.where(neg, ~u, u | jnp.uint32(0x80000000))


def _select_kernel(kk_ref, prow_ref, mm_out, pcol_ref, rank_ref, key_ref):
    f32 = jnp.float32
    p_row = prow_ref[0]

    pcol_ref[...] = _to_col(p_row)
    col_i = jax.lax.broadcasted_iota(jnp.int32, (1, _P), 1)
    ch_iota = jax.lax.broadcasted_iota(jnp.int32, (_CH, 1), 0)

    def rank_body(c, acc):
        p_j = pcol_ref[pl.ds(c * _CH, _CH), :]
        j_j = c * _CH + ch_iota
        gt = (p_j > p_row).astype(f32)
        tie = jnp.logical_and(p_j == p_row, j_j < col_i).astype(f32)
        return acc + jnp.sum(gt + tie, axis=0, keepdims=True)

    rank_row = jax.lax.fori_loop(0, _P // _CH, rank_body,
                                 jnp.zeros((1, _P), f32))
    rank_ref[...] = rank_row

    ch_row_iota = jax.lax.broadcasted_iota(jnp.int32, (_TCH, 1), 0)

    def sel_body(tc, carry):
        t_iota_c = (tc * _TCH + ch_row_iota).astype(f32)

        def tk_body(c, acc):
            rank_c = rank_ref[:, pl.ds(c * _CK, _CK)]
            ohc_c = (rank_c == t_iota_c).astype(f32)
            return acc + _dot_x(ohc_c, kk_ref[0, pl.ds(c * _CK, _CK), :])

        tk = jax.lax.fori_loop(0, _P // _CK, tk_body,
                               jnp.zeros((_TCH, _D), f32))
        tnrm = jnp.sqrt(jnp.sum(tk * tk, axis=1, keepdims=True))
        tkn = tk / jnp.maximum(tnrm, f32(1e-12))

        def sim_body(c, carry2):
            kk_c = kk_ref[0, pl.ds(c * _CK, _CK), :]
            ncol = jnp.sqrt(jnp.sum(kk_c * kk_c, axis=1, keepdims=True))
            kn_c = kk_c / jnp.maximum(ncol, f32(1e-12))
            part = _dot_t(tkn, kn_c)
            rank_c = rank_ref[:, pl.ds(c * _CK, _CK)]
            sim_c = jnp.where(rank_c < f32(_T), f32(_NEG), part)
            key_ref[:, pl.ds(c * _CK, _CK)] = _sort_key(sim_c)
            return carry2

        jax.lax.fori_loop(0, _P // _CK, sim_body, 0)
        key = key_ref[...]

        def bs_body(_, lohi):
            lo, hi = lohi
            mid = lo + ((hi - lo + jnp.uint32(1)) >> jnp.uint32(1))
            cnt = jnp.sum((key >= mid).astype(f32), axis=1, keepdims=True)
            ge = cnt >= f32(_KPT)
            return (jnp.where(ge, mid, lo),
                    jnp.where(ge, hi, mid - jnp.uint32(1)))

        lo0 = jnp.zeros((_TCH, 1), jnp.uint32)
        hi0 = jnp.full((_TCH, 1), 0xFFFFFFFE, jnp.uint32)
        lo, _ = jax.lax.fori_loop(0, 34, bs_body, (lo0, hi0))

        w = (key >= lo).astype(f32) * p_row
        wsum = jnp.sum(w, axis=1, keepdims=True)
        ohc = (rank_row == t_iota_c).astype(f32)
        mm_out[0, pl.ds(tc * _TCH, _TCH), :] = (
            ohc + w / (wsum + f32(1e-6)))
        return carry

    jax.lax.fori_loop(0, _T // _TCH, sel_body, 0)


def _merge_kernel(mm_ref, tok_ref, out_ref):
    j = pl.program_id(1)
    part = _dot_x(mm_ref[0], tok_ref[0])

    @pl.when(j == 0)
    def _init():
        out_ref[0] = part

    @pl.when(j != 0)
    def _acc():
        out_ref[0] = out_ref[0] + part


def _core(patch_keys, p_patch_row, patch_features, interpret=False):
    f32 = jnp.float32
    mm = pl.pallas_call(
        _select_kernel,
        grid=(_B,),
        in_specs=[
            pl.BlockSpec((1, _P, _D), lambda b: (b, 0, 0)),
            pl.BlockSpec((1, 1, _P), lambda b: (b, 0, 0)),
        ],
        out_specs=pl.BlockSpec((1, _T, _P), lambda b: (b, 0, 0)),
        out_shape=jax.ShapeDtypeStruct((_B, _T, _P), f32),
        scratch_shapes=[
            pltpu.VMEM((_P, 1), f32),
            pltpu.VMEM((1, _P), f32),
            pltpu.VMEM((_TCH, _P), jnp.uint32),
        ],
        compiler_params=pltpu.CompilerParams(
            dimension_semantics=("arbitrary",),
            vmem_limit_bytes=128 * 1024 * 1024),
        interpret=interpret,
    )(patch_keys, p_patch_row)

    merged = pl.pallas_call(
        _merge_kernel,
        grid=(_B, _P // _MT),
        in_specs=[
            pl.BlockSpec((1, _T, _MT), lambda b, j: (b, 0, j)),
            pl.BlockSpec((1, _MT, _D), lambda b, j: (b, j, 0)),
        ],
        out_specs=pl.BlockSpec((1, _T, _D), lambda b, j: (b, 0, 0)),
        out_shape=jax.ShapeDtypeStruct((_B, _T, _D), f32),
        compiler_params=pltpu.CompilerParams(
            dimension_semantics=("parallel", "arbitrary"),
            vmem_limit_bytes=128 * 1024 * 1024),
        interpret=interpret,
    )(mm, patch_features)
    return merged


def kernel(pixel_values, patch_coords, cls_token, W_in, W_coord, W_q, W_k):
    tokens = pixel_values @ W_in + patch_coords @ W_coord
    cls = jnp.broadcast_to(cls_token, (tokens.shape[0], 1, tokens.shape[-1]))
    toks = jnp.concatenate([cls, tokens], axis=1)
    q = toks @ W_q
    k = toks @ W_k

    cls_q = q[:, 0:1, :]
    attn = (cls_q @ jnp.swapaxes(k, -1, -2)) / math.sqrt(k.shape[-1])
    cls_attn_to_all = jax.nn.softmax(attn, axis=-1)[:, 0, :]
    p_patch = cls_attn_to_all[:, 1:]

    merged = _core(k[:, 1:], p_patch[:, None, :], tokens)
    return jnp.concatenate([toks[:, 0:1, :], merged], axis=1)
